# baseline (speedup 1.0000x reference)
import functools

import jax
import jax.numpy as jnp
from jax import lax
from jax.experimental import pallas as pl
from jax.experimental.pallas import tpu as pltpu
from jax.experimental.pallas import tpu_sc as plsc

B = 16384
IB = 1024
NBITS = 10
RS = 1 << NBITS
NC = 2
NS = 16
NW = NC * NS
CHUNK = B // NW
NG = CHUNK // 16
IDXW = 128
NJ = CHUNK // IDXW
SL = RS // NS

_mesh = plsc.VectorSubcoreMesh(core_axis_name="c", subcore_axis_name="s")


@functools.partial(
    pl.kernel,
    mesh=_mesh,
    compiler_params=pltpu.CompilerParams(needs_layout_passes=False),
    out_type=(
        jax.ShapeDtypeStruct((NC * RS,), jnp.int32),
        jax.ShapeDtypeStruct((B,), jnp.int32),
    ),
    scratch_types=[
        pltpu.VMEM((16,), jnp.int32),
        pltpu.VMEM((CHUNK,), jnp.int32),
        pltpu.VMEM((NBITS * CHUNK,), jnp.int32),
        pltpu.VMEM((NBITS * CHUNK,), jnp.int32),
        pltpu.VMEM((CHUNK,), jnp.int32),
        pltpu.VMEM((CHUNK,), jnp.int32),
        pltpu.VMEM((CHUNK,), jnp.int32),
        pltpu.VMEM((16 * RS,), jnp.int32),
        pltpu.VMEM((RS,), jnp.int32),
        pltpu.VMEM((NS, SL), jnp.int32),
        pltpu.VMEM((SL,), jnp.int32),
        pltpu.VMEM_SHARED((NS * RS,), jnp.int32),
        pltpu.SemaphoreType.DMA,
        pltpu.SemaphoreType.DMA,
    ],
)
def _phase1(a_hbm, b_hbm, av_hbm, bv_hbm, map_hbm,
            wt_hbm, addr_hbm,
            map_v, ramp, idxb, colsf, av, bv, addrs, tbl, loc, mrg, msl,
            shared, gsem, lsem):
    c = lax.axis_index("c")
    s = lax.axis_index("s")
    wid = s * NC + c
    base = wid * CHUNK

    pltpu.sync_copy(map_hbm, map_v.at[pl.ds(0, NBITS)])
    av_cp = pltpu.async_copy(av_hbm.at[pl.ds(base, CHUNK)], av, lsem)
    bv_cp = pltpu.async_copy(bv_hbm.at[pl.ds(base, CHUNK)], bv, lsem)

    lanes = lax.iota(jnp.int32, 16)
    mv = map_v[...]
    gs = [jnp.max(jnp.where(lanes == k, mv, -1)) for k in range(NBITS)]

    def bramp(i, _):
        for u in range(4):
            iv = base + i * 64 + u * 16 + lanes
            ramp[pl.ds(i * 64 + u * 16, 16)] = (iv >> 3) * 8192 + (iv & 7) * 128
        return 0

    lax.fori_loop(0, NG // 4, bramp, 0)

    for k in range(NBITS):
        g = gs[k]
        col = jnp.where(g < IB, g, g - IB)
        coff = (col >> 7) * 1024 + (col & 127)

        def bidx(i, _):
            for u in range(4):
                o = i * 64 + u * 16
                idxb[pl.ds(k * CHUNK + o, 16)] = ramp[pl.ds(o, 16)] + coff
            return 0

        lax.fori_loop(0, NG // 4, bidx, 0)

    for k in range(NBITS):
        for j in range(NJ):
            off = k * CHUNK + j * IDXW
            idx_ref = idxb.at[pl.ds(off, IDXW)]
            dst = colsf.at[pl.ds(off, IDXW)]

            @pl.when(gs[k] < IB)
            def _():
                pltpu.async_copy(a_hbm.at[idx_ref], dst, gsem)

            @pl.when(gs[k] >= IB)
            def _():
                pltpu.async_copy(b_hbm.at[idx_ref], dst, gsem)

    neg1 = jnp.full((16,), -1, jnp.int32)

    def init_body(i, _):
        for u in range(8):
            tbl[pl.ds(i * 128 + u * 16, 16)] = neg1
        return 0

    lax.fori_loop(0, 16 * RS // 128, init_body, 0)

    for k in range(NBITS):
        for j in range(NJ):
            off = k * CHUNK + j * IDXW
            pltpu.make_async_copy(
                a_hbm.at[pl.ds(0, IDXW)], colsf.at[pl.ds(off, IDXW)],
                gsem).wait()
    av_cp.wait()
    bv_cp.wait()

    def grp(g, _):
        ad = jnp.zeros((16,), jnp.int32)
        for k in range(NBITS):
            ad = ad + colsf[pl.ds(k * CHUNK + g * 16, 16)] * (1 << k)
        addrs[pl.ds(g * 16, 16)] = ad
        t = jnp.where(av[pl.ds(g * 16, 16)] < bv[pl.ds(g * 16, 16)], 1, 0)
        w = 2 * (base + g * 16 + lanes) + t
        plsc.store_scatter(tbl, [lanes * RS + ad], w)
        return 0

    lax.fori_loop(0, NG, grp, 0)

    def merge_lanes(cc, _):
        m = tbl[pl.ds(cc * 16, 16)]
        for l in range(1, 16):
            m = jnp.maximum(m, tbl[pl.ds(l * RS + cc * 16, 16)])
        loc[pl.ds(cc * 16, 16)] = m
        return 0

    lax.fori_loop(0, RS // 16, merge_lanes, 0)

    pltpu.sync_copy(loc, shared.at[pl.ds(s * RS, RS)])
    plsc.subcore_barrier()
    slice_cps = [
        pltpu.async_copy(shared.at[pl.ds(t * RS + s * SL, SL)], mrg.at[t],
                         lsem)
        for t in range(NS)
    ]
    for cp in slice_cps:
        cp.wait()

    for u in range(SL // 16):
        m = mrg[0, pl.ds(u * 16, 16)]
        for t in range(1, NS):
            m = jnp.maximum(m, mrg[t, pl.ds(u * 16, 16)])
        msl[pl.ds(u * 16, 16)] = m

    pltpu.sync_copy(msl, wt_hbm.at[pl.ds(c * RS + s * SL, SL)])
    pltpu.sync_copy(addrs, addr_hbm.at[pl.ds(base, CHUNK)])


@functools.partial(
    pl.kernel,
    mesh=_mesh,
    compiler_params=pltpu.CompilerParams(needs_layout_passes=False),
    out_type=jax.ShapeDtypeStruct((B,), jnp.float32),
    scratch_types=[
        pltpu.VMEM((NC * RS,), jnp.int32),
        pltpu.VMEM((RS,), jnp.int32),
        pltpu.VMEM((CHUNK,), jnp.int32),
        pltpu.VMEM((CHUNK,), jnp.float32),
    ],
)
def _phase2(wt_hbm, addr_hbm, out_hbm, wt, jwin, addrs, outs):
    c = lax.axis_index("c")
    s = lax.axis_index("s")
    base = (s * NC + c) * CHUNK

    pltpu.sync_copy(wt_hbm, wt)
    pltpu.sync_copy(addr_hbm.at[pl.ds(base, CHUNK)], addrs)

    def merge_col(cc, _):
        m = jnp.maximum(wt[pl.ds(cc * 16, 16)], wt[pl.ds(RS + cc * 16, 16)])
        jwin[pl.ds(cc * 16, 16)] = m
        return 0

    lax.fori_loop(0, RS // 16, merge_col, 0)

    def ogrp(g, _):
        ad = addrs[pl.ds(g * 16, 16)]
        wv = plsc.load_gather(jwin, [ad])
        outs[pl.ds(g * 16, 16)] = (wv & 1).astype(jnp.float32)
        return 0

    lax.fori_loop(0, NG, ogrp, 0)

    pltpu.sync_copy(outs, out_hbm.at[pl.ds(base, CHUNK)])


def _flat_tiled_view(x):
    return x.reshape(B // 8, 8, IB // 128, 128).transpose(0, 2, 1, 3).reshape(-1)


def kernel(a, b, a_val, b_val, mapping, memory):
    del memory
    wt, addrs = _phase1(_flat_tiled_view(a), _flat_tiled_view(b),
                        a_val.astype(jnp.int32), b_val.astype(jnp.int32),
                        mapping.astype(jnp.int32))
    return _phase2(wt, addrs)


# speedup vs baseline: 1.0287x; 1.0287x over previous
import functools

import jax
import jax.numpy as jnp
from jax import lax
from jax.experimental import pallas as pl
from jax.experimental.pallas import tpu as pltpu
from jax.experimental.pallas import tpu_sc as plsc

B = 16384
IB = 1024
NBITS = 10
RS = 1 << NBITS
NC = 2
NS = 16
NW = NC * NS
CHUNK = B // NW
NG = CHUNK // 16
IDXW = 128
NJ = CHUNK // IDXW
SL = RS // NS

_mesh = plsc.VectorSubcoreMesh(core_axis_name="c", subcore_axis_name="s")


@functools.partial(
    pl.kernel,
    mesh=_mesh,
    compiler_params=pltpu.CompilerParams(needs_layout_passes=False),
    out_type=(
        jax.ShapeDtypeStruct((NC * RS,), jnp.int32),
        jax.ShapeDtypeStruct((B,), jnp.int32),
    ),
    scratch_types=[
        pltpu.VMEM((16,), jnp.int32),
        pltpu.VMEM((CHUNK,), jnp.int32),
        pltpu.VMEM((NBITS * CHUNK,), jnp.int32),
        pltpu.VMEM((NBITS * CHUNK,), jnp.int32),
        pltpu.VMEM((CHUNK,), jnp.int32),
        pltpu.VMEM((CHUNK,), jnp.int32),
        pltpu.VMEM((CHUNK,), jnp.int32),
        pltpu.VMEM((8 * RS,), jnp.int32),
        pltpu.VMEM((RS,), jnp.int32),
        pltpu.VMEM((NS, SL), jnp.int32),
        pltpu.VMEM((SL,), jnp.int32),
        pltpu.VMEM_SHARED((NS * RS,), jnp.int32),
        pltpu.SemaphoreType.DMA,
        pltpu.SemaphoreType.DMA,
    ],
)
def _phase1(a_hbm, b_hbm, av_hbm, bv_hbm, map_hbm,
            wt_hbm, addr_hbm,
            map_v, ramp, idxb, colsf, av, bv, addrs, tbl, loc, mrg, msl,
            shared, gsem, lsem):
    c = lax.axis_index("c")
    s = lax.axis_index("s")
    wid = s * NC + c
    base = wid * CHUNK

    pltpu.sync_copy(map_hbm, map_v.at[pl.ds(0, NBITS)])
    av_cp = pltpu.async_copy(av_hbm.at[pl.ds(base, CHUNK)], av, lsem)
    bv_cp = pltpu.async_copy(bv_hbm.at[pl.ds(base, CHUNK)], bv, lsem)

    lanes = lax.iota(jnp.int32, 16)
    mv = map_v[...]
    gs = [jnp.max(jnp.where(lanes == k, mv, -1)) for k in range(NBITS)]

    def bramp(i, _):
        for u in range(4):
            iv = base + i * 64 + u * 16 + lanes
            ramp[pl.ds(i * 64 + u * 16, 16)] = (iv >> 3) * 8192 + (iv & 7) * 128
        return 0

    lax.fori_loop(0, NG // 4, bramp, 0)

    for k in range(NBITS):
        g = gs[k]
        col = jnp.where(g < IB, g, g - IB)
        coff = (col >> 7) * 1024 + (col & 127)

        def bidx(i, _):
            for u in range(4):
                o = i * 64 + u * 16
                idxb[pl.ds(k * CHUNK + o, 16)] = ramp[pl.ds(o, 16)] + coff
            return 0

        lax.fori_loop(0, NG // 4, bidx, 0)

    for k in range(NBITS):
        for j in range(NJ):
            off = k * CHUNK + j * IDXW
            idx_ref = idxb.at[pl.ds(off, IDXW)]
            dst = colsf.at[pl.ds(off, IDXW)]

            @pl.when(gs[k] < IB)
            def _():
                pltpu.async_copy(a_hbm.at[idx_ref], dst, gsem)

            @pl.when(gs[k] >= IB)
            def _():
                pltpu.async_copy(b_hbm.at[idx_ref], dst, gsem)

    neg1 = jnp.full((16,), -1, jnp.int32)

    def init_body(i, _):
        for u in range(8):
            tbl[pl.ds(i * 128 + u * 16, 16)] = neg1
        return 0

    lax.fori_loop(0, 8 * RS // 128, init_body, 0)

    pltpu.make_async_copy(
        a_hbm.at[pl.ds(0, NBITS * CHUNK)], colsf, gsem).wait()
    av_cp.wait()
    bv_cp.wait()

    tsel = jnp.minimum(lanes, 15 - lanes) * RS
    low8 = lanes < 8

    def grp(g, _):
        ad = jnp.zeros((16,), jnp.int32)
        for k in range(NBITS):
            ad = ad + colsf[pl.ds(k * CHUNK + g * 16, 16)] * (1 << k)
        addrs[pl.ds(g * 16, 16)] = ad
        t = jnp.where(av[pl.ds(g * 16, 16)] < bv[pl.ds(g * 16, 16)], 1, 0)
        w = 2 * (base + g * 16 + lanes) + t
        keep = jnp.logical_not((ad == lax.rev(ad, (0,))) & low8)
        plsc.store_scatter(tbl, [tsel + ad], w, mask=keep)
        return 0

    lax.fori_loop(0, NG, grp, 0)

    def merge_lanes(cc, _):
        m = tbl[pl.ds(cc * 16, 16)]
        for l in range(1, 8):
            m = jnp.maximum(m, tbl[pl.ds(l * RS + cc * 16, 16)])
        loc[pl.ds(cc * 16, 16)] = m
        return 0

    lax.fori_loop(0, RS // 16, merge_lanes, 0)

    pltpu.sync_copy(loc, shared.at[pl.ds(s * RS, RS)])
    plsc.subcore_barrier()
    slice_cps = [
        pltpu.async_copy(shared.at[pl.ds(t * RS + s * SL, SL)], mrg.at[t],
                         lsem)
        for t in range(NS)
    ]
    for cp in slice_cps:
        cp.wait()

    for u in range(SL // 16):
        m = mrg[0, pl.ds(u * 16, 16)]
        for t in range(1, NS):
            m = jnp.maximum(m, mrg[t, pl.ds(u * 16, 16)])
        msl[pl.ds(u * 16, 16)] = m

    pltpu.sync_copy(msl, wt_hbm.at[pl.ds(c * RS + s * SL, SL)])
    pltpu.sync_copy(addrs, addr_hbm.at[pl.ds(base, CHUNK)])


@functools.partial(
    pl.kernel,
    mesh=_mesh,
    compiler_params=pltpu.CompilerParams(needs_layout_passes=False),
    out_type=jax.ShapeDtypeStruct((B,), jnp.float32),
    scratch_types=[
        pltpu.VMEM((NC * RS,), jnp.int32),
        pltpu.VMEM((RS,), jnp.int32),
        pltpu.VMEM((CHUNK,), jnp.int32),
        pltpu.VMEM((CHUNK,), jnp.float32),
    ],
)
def _phase2(wt_hbm, addr_hbm, out_hbm, wt, jwin, addrs, outs):
    c = lax.axis_index("c")
    s = lax.axis_index("s")
    base = (s * NC + c) * CHUNK

    pltpu.sync_copy(wt_hbm, wt)
    pltpu.sync_copy(addr_hbm.at[pl.ds(base, CHUNK)], addrs)

    def merge_col(cc, _):
        m = jnp.maximum(wt[pl.ds(cc * 16, 16)], wt[pl.ds(RS + cc * 16, 16)])
        jwin[pl.ds(cc * 16, 16)] = m
        return 0

    lax.fori_loop(0, RS // 16, merge_col, 0)

    def ogrp(g, _):
        ad = addrs[pl.ds(g * 16, 16)]
        wv = plsc.load_gather(jwin, [ad])
        outs[pl.ds(g * 16, 16)] = (wv & 1).astype(jnp.float32)
        return 0

    lax.fori_loop(0, NG, ogrp, 0)

    pltpu.sync_copy(outs, out_hbm.at[pl.ds(base, CHUNK)])


def _flat_tiled_view(x):
    return x.reshape(B // 8, 8, IB // 128, 128).transpose(0, 2, 1, 3).reshape(-1)


def kernel(a, b, a_val, b_val, mapping, memory):
    del memory
    wt, addrs = _phase1(_flat_tiled_view(a), _flat_tiled_view(b),
                        a_val.astype(jnp.int32), b_val.astype(jnp.int32),
                        mapping.astype(jnp.int32))
    return _phase2(wt, addrs)


# speedup vs baseline: 1.0545x; 1.0251x over previous
import functools

import jax
import jax.numpy as jnp
from jax import lax
from jax.experimental import pallas as pl
from jax.experimental.pallas import tpu as pltpu
from jax.experimental.pallas import tpu_sc as plsc

B = 16384
IB = 1024
NBITS = 10
RS = 1 << NBITS
NC = 2
NS = 16
NW = NC * NS
CHUNK = B // NW
NG = CHUNK // 16
IDXW = 128
NJ = CHUNK // IDXW
SL = RS // NS

_mesh = plsc.VectorSubcoreMesh(core_axis_name="c", subcore_axis_name="s")


@functools.partial(
    pl.kernel,
    mesh=_mesh,
    compiler_params=pltpu.CompilerParams(needs_layout_passes=False),
    out_type=(
        jax.ShapeDtypeStruct((NC * RS,), jnp.int32),
        jax.ShapeDtypeStruct((B,), jnp.int32),
    ),
    scratch_types=[
        pltpu.VMEM((16,), jnp.int32),
        pltpu.VMEM((CHUNK,), jnp.int32),
        pltpu.VMEM((NBITS * CHUNK,), jnp.int32),
        pltpu.VMEM((NBITS * CHUNK,), jnp.int32),
        pltpu.VMEM((CHUNK,), jnp.int32),
        pltpu.VMEM((CHUNK,), jnp.int32),
        pltpu.VMEM((CHUNK,), jnp.int32),
        pltpu.VMEM((8 * RS,), jnp.int32),
        pltpu.VMEM((RS,), jnp.int32),
        pltpu.VMEM((NS, SL), jnp.int32),
        pltpu.VMEM((SL,), jnp.int32),
        pltpu.VMEM_SHARED((NS * RS,), jnp.int32),
        pltpu.SemaphoreType.DMA,
        pltpu.SemaphoreType.DMA,
    ],
)
def _phase1(a_hbm, b_hbm, av_hbm, bv_hbm, map_hbm,
            wt_hbm, addr_hbm,
            map_v, ramp, idxb, colsf, av, bv, addrs, tbl, loc, mrg, msl,
            shared, gsem, lsem):
    c = lax.axis_index("c")
    s = lax.axis_index("s")
    wid = s * NC + c
    base = wid * CHUNK

    pltpu.sync_copy(map_hbm, map_v.at[pl.ds(0, NBITS)])
    av_cp = pltpu.async_copy(av_hbm.at[pl.ds(base, CHUNK)], av, lsem)
    bv_cp = pltpu.async_copy(bv_hbm.at[pl.ds(base, CHUNK)], bv, lsem)

    lanes = lax.iota(jnp.int32, 16)
    mv = map_v[...]
    gs = [jnp.max(jnp.where(lanes == k, mv, -1)) for k in range(NBITS)]

    def bramp(i, _):
        for u in range(4):
            iv = base + i * 64 + u * 16 + lanes
            ramp[pl.ds(i * 64 + u * 16, 16)] = (iv >> 3) * 8192 + (iv & 7) * 128
        return 0

    lax.fori_loop(0, NG // 4, bramp, 0)

    for k in range(NBITS):
        g = gs[k]
        col = jnp.where(g < IB, g, g - IB)
        coff = (col >> 7) * 1024 + (col & 127)

        def bidx(i, _):
            for u in range(4):
                o = i * 64 + u * 16
                idxb[pl.ds(k * CHUNK + o, 16)] = ramp[pl.ds(o, 16)] + coff
            return 0

        lax.fori_loop(0, NG // 4, bidx, 0)

        for j in range(NJ):
            off = k * CHUNK + j * IDXW
            idx_ref = idxb.at[pl.ds(off, IDXW)]
            dst = colsf.at[pl.ds(off, IDXW)]

            @pl.when(g < IB)
            def _():
                pltpu.async_copy(a_hbm.at[idx_ref], dst, gsem)

            @pl.when(g >= IB)
            def _():
                pltpu.async_copy(b_hbm.at[idx_ref], dst, gsem)

    neg1 = jnp.full((16,), -1, jnp.int32)

    def init_body(i, _):
        for u in range(8):
            tbl[pl.ds(i * 128 + u * 16, 16)] = neg1
        return 0

    lax.fori_loop(0, 8 * RS // 128, init_body, 0)

    pltpu.make_async_copy(
        a_hbm.at[pl.ds(0, NBITS * CHUNK)], colsf, gsem).wait()
    av_cp.wait()
    bv_cp.wait()

    tsel = jnp.minimum(lanes, 15 - lanes) * RS
    low8 = lanes < 8

    def grp(g, _):
        ad = jnp.zeros((16,), jnp.int32)
        for k in range(NBITS):
            ad = ad + colsf[pl.ds(k * CHUNK + g * 16, 16)] * (1 << k)
        addrs[pl.ds(g * 16, 16)] = ad
        t = jnp.where(av[pl.ds(g * 16, 16)] < bv[pl.ds(g * 16, 16)], 1, 0)
        w = 2 * (base + g * 16 + lanes) + t
        keep = jnp.logical_not((ad == lax.rev(ad, (0,))) & low8)
        plsc.store_scatter(tbl, [tsel + ad], w, mask=keep)
        return 0

    lax.fori_loop(0, NG, grp, 0)

    addr_cp = pltpu.async_copy(addrs, addr_hbm.at[pl.ds(base, CHUNK)], gsem)

    def merge_lanes(cc, _):
        m = tbl[pl.ds(cc * 16, 16)]
        for l in range(1, 8):
            m = jnp.maximum(m, tbl[pl.ds(l * RS + cc * 16, 16)])
        loc[pl.ds(cc * 16, 16)] = m
        return 0

    lax.fori_loop(0, RS // 16, merge_lanes, 0)

    pltpu.sync_copy(loc, shared.at[pl.ds(s * RS, RS)])
    plsc.subcore_barrier()
    slice_cps = [
        pltpu.async_copy(shared.at[pl.ds(t * RS + s * SL, SL)], mrg.at[t],
                         lsem)
        for t in range(NS)
    ]
    for cp in slice_cps:
        cp.wait()

    for u in range(SL // 16):
        m = mrg[0, pl.ds(u * 16, 16)]
        for t in range(1, NS):
            m = jnp.maximum(m, mrg[t, pl.ds(u * 16, 16)])
        msl[pl.ds(u * 16, 16)] = m

    pltpu.sync_copy(msl, wt_hbm.at[pl.ds(c * RS + s * SL, SL)])
    addr_cp.wait()


@functools.partial(
    pl.kernel,
    mesh=_mesh,
    compiler_params=pltpu.CompilerParams(needs_layout_passes=False),
    out_type=jax.ShapeDtypeStruct((B,), jnp.float32),
    scratch_types=[
        pltpu.VMEM((NC * RS,), jnp.int32),
        pltpu.VMEM((CHUNK,), jnp.int32),
        pltpu.VMEM((CHUNK,), jnp.float32),
        pltpu.SemaphoreType.DMA,
    ],
)
def _phase2(wt_hbm, addr_hbm, out_hbm, wt, addrs, outs, lsem):
    c = lax.axis_index("c")
    s = lax.axis_index("s")
    base = (s * NC + c) * CHUNK

    wt_cp = pltpu.async_copy(wt_hbm, wt, lsem)
    pltpu.sync_copy(addr_hbm.at[pl.ds(base, CHUNK)], addrs)
    wt_cp.wait()

    def ogrp(g, _):
        ad = addrs[pl.ds(g * 16, 16)]
        wv = jnp.maximum(plsc.load_gather(wt, [ad]),
                         plsc.load_gather(wt, [RS + ad]))
        outs[pl.ds(g * 16, 16)] = (wv & 1).astype(jnp.float32)
        return 0

    lax.fori_loop(0, NG, ogrp, 0)

    pltpu.sync_copy(outs, out_hbm.at[pl.ds(base, CHUNK)])


def _flat_tiled_view(x):
    return x.reshape(B // 8, 8, IB // 128, 128).transpose(0, 2, 1, 3).reshape(-1)


def kernel(a, b, a_val, b_val, mapping, memory):
    del memory
    wt, addrs = _phase1(_flat_tiled_view(a), _flat_tiled_view(b),
                        a_val.astype(jnp.int32), b_val.astype(jnp.int32),
                        mapping.astype(jnp.int32))
    return _phase2(wt, addrs)
